# Initial kernel scaffold; baseline (speedup 1.0000x reference)
import functools

import jax
import jax.numpy as jnp
from jax import lax
from jax.experimental import pallas as pl
from jax.experimental.pallas import tpu as pltpu
from jax.experimental.pallas import tpu_sc as plsc

E = 6_400_000
N = 100_000
NW = 32
EPW = E // NW
C = 2_000
NCH = EPW // C
G = C // 16
NROWS = 896
STRIPE = NROWS // 16

_mesh = plsc.VectorSubcoreMesh(core_axis_name="c", subcore_axis_name="s")

_ZERO16 = jnp.zeros((16,), jnp.float32)


def _worker(c, s):
    return c * 16 + s


@functools.partial(
    pl.kernel,
    out_type=jax.ShapeDtypeStruct((2, NROWS, 128), jnp.float32),
    mesh=_mesh,
    scratch_types=[
        pltpu.VMEM((NROWS, 128), jnp.float32),
        pltpu.VMEM((C,), jnp.int32),
        pltpu.VMEM((C,), jnp.float32),
        pltpu.VMEM((NROWS // 128, 128), jnp.int32),
        pltpu.VMEM_SHARED((NROWS, 128), jnp.float32),
    ],
)
def _segment_sum(ei, w, rows, out, denom_v, idx_v, w_v, rows_v, shared):
    c = lax.axis_index("c")
    s = lax.axis_index("s")
    wid = _worker(c, s)

    def zb(r, carry):
        for cc in range(8):
            denom_v[r, pl.ds(cc * 16, 16)] = _ZERO16
        return carry
    lax.fori_loop(0, NROWS, zb, 0)

    pltpu.sync_copy(denom_v.at[pl.ds(s * STRIPE, STRIPE)],
                    shared.at[pl.ds(s * STRIPE, STRIPE)])
    pltpu.sync_copy(rows, rows_v)

    def chunk(j, carry):
        base = wid * EPW + j * C
        pltpu.sync_copy(ei.at[1, pl.ds(base, C)], idx_v)
        pltpu.sync_copy(w.at[pl.ds(base, C)], w_v)

        def grp(g, cc):
            o = g * 16
            idx = idx_v[pl.ds(o, 16)]
            ew = jnp.exp(w_v[pl.ds(o, 16)])
            row = lax.shift_right_logical(idx, 7)
            col = lax.bitwise_and(idx, 127)
            plsc.addupdate_scatter(denom_v, [row, col], ew)
            return cc
        lax.fori_loop(0, G, grp, 0)
        return carry
    lax.fori_loop(0, NCH, chunk, 0)

    plsc.subcore_barrier()

    for q in range(NROWS // 128):
        pltpu.sync_copy(denom_v.at[pl.ds(q * 128, 128)],
                        shared.at[rows_v.at[q]], add=True)

    plsc.subcore_barrier()

    pltpu.sync_copy(shared.at[pl.ds(s * STRIPE, STRIPE)],
                    out.at[c, pl.ds(s * STRIPE, STRIPE)])


@functools.partial(
    pl.kernel,
    out_type=jax.ShapeDtypeStruct((E,), jnp.float32),
    mesh=_mesh,
    scratch_types=[
        pltpu.VMEM((NROWS, 128), jnp.float32),
        pltpu.VMEM((C,), jnp.int32),
        pltpu.VMEM((C,), jnp.float32),
        pltpu.VMEM((C,), jnp.float32),
    ],
)
def _normalize(ei, w, denom, out, denom_v, idx_v, w_v, out_v):
    c = lax.axis_index("c")
    s = lax.axis_index("s")
    wid = _worker(c, s)

    pltpu.sync_copy(denom, denom_v)

    def chunk(j, carry):
        base = wid * EPW + j * C
        pltpu.sync_copy(ei.at[1, pl.ds(base, C)], idx_v)
        pltpu.sync_copy(w.at[pl.ds(base, C)], w_v)

        def grp(g, cc):
            o = g * 16
            idx = idx_v[pl.ds(o, 16)]
            ew = jnp.exp(w_v[pl.ds(o, 16)])
            row = lax.shift_right_logical(idx, 7)
            col = lax.bitwise_and(idx, 127)
            d = plsc.load_gather(denom_v, [row, col])
            out_v[pl.ds(o, 16)] = ew / d
            return cc
        lax.fori_loop(0, G, grp, 0)
        pltpu.sync_copy(out_v, out.at[pl.ds(base, C)])
        return carry
    lax.fori_loop(0, NCH, chunk, 0)


def kernel(edge_index, edge_weight):
    rows = jnp.arange(NROWS, dtype=jnp.int32).reshape(NROWS // 128, 128)
    partial = _segment_sum(edge_index, edge_weight, rows)
    denom = partial[0] + partial[1]
    wnorm = _normalize(edge_index, edge_weight, denom)
    return (edge_index, wnorm)


# speedup vs baseline: 93.6220x; 93.6220x over previous
import functools

import jax
import jax.numpy as jnp
from jax import lax
from jax.experimental import pallas as pl
from jax.experimental.pallas import tpu as pltpu
from jax.experimental.pallas import tpu_sc as plsc

E = 6_400_000
N = 100_000
NW = 32
EPW = E // NW
C = 2_000
NCH = EPW // C
G = C // 16
NTAB = 114_688

_mesh = plsc.VectorSubcoreMesh(core_axis_name="c", subcore_axis_name="s")


def _worker():
    return lax.axis_index("c") * 16 + lax.axis_index("s")


@functools.partial(
    pl.kernel,
    out_type=jax.ShapeDtypeStruct((NW * NTAB,), jnp.float32),
    mesh=_mesh,
    scratch_types=[
        pltpu.VMEM((NTAB,), jnp.float32),
        pltpu.VMEM((C,), jnp.int32),
        pltpu.VMEM((C,), jnp.float32),
    ],
    compiler_params=pltpu.CompilerParams(needs_layout_passes=False),
)
def _segment_sum(seg, w, out, denom_v, idx_v, w_v):
    wid = _worker()

    zero16 = jnp.zeros((16,), jnp.float32)

    def zb(r, carry):
        for u in range(8):
            denom_v[pl.ds(r * 128 + u * 16, 16)] = zero16
        return carry
    lax.fori_loop(0, NTAB // 128, zb, 0)

    def chunk(j, carry):
        base = wid * EPW + j * C
        pltpu.sync_copy(seg.at[pl.ds(base, C)], idx_v)
        pltpu.sync_copy(w.at[pl.ds(base, C)], w_v)

        def grp(g, cc):
            o = g * 16
            idx = idx_v[pl.ds(o, 16)]
            ew = jnp.exp(w_v[pl.ds(o, 16)])
            plsc.addupdate_scatter(denom_v, [idx], ew)
            return cc
        lax.fori_loop(0, G, grp, 0)
        return carry
    lax.fori_loop(0, NCH, chunk, 0)

    pltpu.sync_copy(denom_v, out.at[pl.ds(wid * NTAB, NTAB)])


@functools.partial(
    pl.kernel,
    out_type=jax.ShapeDtypeStruct((E,), jnp.float32),
    mesh=_mesh,
    scratch_types=[
        pltpu.VMEM((NTAB,), jnp.float32),
        pltpu.VMEM((C,), jnp.int32),
        pltpu.VMEM((C,), jnp.float32),
        pltpu.VMEM((C,), jnp.float32),
    ],
    compiler_params=pltpu.CompilerParams(needs_layout_passes=False),
)
def _normalize(seg, w, denom, out, denom_v, idx_v, w_v, out_v):
    wid = _worker()

    pltpu.sync_copy(denom, denom_v)

    def chunk(j, carry):
        base = wid * EPW + j * C
        pltpu.sync_copy(seg.at[pl.ds(base, C)], idx_v)
        pltpu.sync_copy(w.at[pl.ds(base, C)], w_v)

        def grp(g, cc):
            o = g * 16
            idx = idx_v[pl.ds(o, 16)]
            ew = jnp.exp(w_v[pl.ds(o, 16)])
            d = plsc.load_gather(denom_v, [idx])
            out_v[pl.ds(o, 16)] = ew / d
            return cc
        lax.fori_loop(0, G, grp, 0)
        pltpu.sync_copy(out_v, out.at[pl.ds(base, C)])
        return carry
    lax.fori_loop(0, NCH, chunk, 0)


def kernel(edge_index, edge_weight):
    seg = edge_index[1]
    partial = _segment_sum(seg, edge_weight)
    denom = jnp.sum(partial.reshape(NW, NTAB), axis=0)
    wnorm = _normalize(seg, edge_weight, denom)
    return (edge_index, wnorm)


# speedup vs baseline: 159.6254x; 1.7050x over previous
import functools

import jax
import jax.numpy as jnp
from jax import lax
from jax.experimental import pallas as pl
from jax.experimental.pallas import tpu as pltpu
from jax.experimental.pallas import tpu_sc as plsc

E = 6_400_000
N = 100_000
NW = 32
EPW = E // NW
NTAB = 114_688

C1 = 4_000
NCH1 = EPW // C1
G1 = C1 // 16
U1 = 5

C2 = 2_000
NCH2 = EPW // C2
G2 = C2 // 16
U2 = 5

_mesh = plsc.VectorSubcoreMesh(core_axis_name="c", subcore_axis_name="s")
_params = pltpu.CompilerParams(needs_layout_passes=False)


def _worker():
    return lax.axis_index("c") * 16 + lax.axis_index("s")


@functools.partial(
    pl.kernel,
    out_type=jax.ShapeDtypeStruct((NW * NTAB,), jnp.float32),
    mesh=_mesh,
    scratch_types=[
        pltpu.VMEM((NTAB,), jnp.float32),
        pltpu.VMEM((C1,), jnp.int32),
        pltpu.VMEM((C1,), jnp.int32),
        pltpu.VMEM((C1,), jnp.float32),
        pltpu.VMEM((C1,), jnp.float32),
        pltpu.SemaphoreType.DMA((2,)),
        pltpu.SemaphoreType.DMA((2,)),
    ],
    compiler_params=_params,
)
def _segment_sum(seg, w, out, denom_v, idx_v0, idx_v1, w_v0, w_v1,
                 isem, wsem):
    wid = _worker()
    idx_b = (idx_v0, idx_v1)
    w_b = (w_v0, w_v1)

    def start(j, b):
        base = wid * EPW + j * C1
        pltpu.async_copy(seg.at[pl.ds(base, C1)], idx_b[b], isem.at[b])
        pltpu.async_copy(w.at[pl.ds(base, C1)], w_b[b], wsem.at[b])

    start(0, 0)
    start(1, 1)

    zero16 = jnp.zeros((16,), jnp.float32)

    def zb(r, carry):
        for u in range(8):
            denom_v[pl.ds(r * 128 + u * 16, 16)] = zero16
        return carry
    lax.fori_loop(0, NTAB // 128, zb, 0)

    def pair(p, carry):
        for b in range(2):
            j = p * 2 + b
            pltpu.make_async_copy(seg.at[pl.ds(0, C1)], idx_b[b],
                                  isem.at[b]).wait()
            pltpu.make_async_copy(w.at[pl.ds(0, C1)], w_b[b],
                                  wsem.at[b]).wait()

            def grp(g, cc):
                for u in range(U1):
                    o = (g * U1 + u) * 16
                    idx = idx_b[b][pl.ds(o, 16)]
                    ew = jnp.exp(w_b[b][pl.ds(o, 16)])
                    plsc.addupdate_scatter(denom_v, [idx], ew)
                return cc
            lax.fori_loop(0, G1 // U1, grp, 0)

            @pl.when(j + 2 < NCH1)
            def _():
                start(j + 2, b)
        return carry
    lax.fori_loop(0, NCH1 // 2, pair, 0)

    pltpu.sync_copy(denom_v, out.at[pl.ds(wid * NTAB, NTAB)])


@functools.partial(
    pl.kernel,
    out_type=jax.ShapeDtypeStruct((E,), jnp.float32),
    mesh=_mesh,
    scratch_types=[
        pltpu.VMEM((NTAB,), jnp.float32),
        pltpu.VMEM((C2,), jnp.int32),
        pltpu.VMEM((C2,), jnp.int32),
        pltpu.VMEM((C2,), jnp.float32),
        pltpu.VMEM((C2,), jnp.float32),
        pltpu.VMEM((C2,), jnp.float32),
        pltpu.VMEM((C2,), jnp.float32),
        pltpu.SemaphoreType.DMA((2,)),
        pltpu.SemaphoreType.DMA((2,)),
        pltpu.SemaphoreType.DMA((2,)),
        pltpu.SemaphoreType.DMA,
    ],
    compiler_params=_params,
)
def _normalize(seg, w, denom, out, denom_v, idx_v0, idx_v1, w_v0, w_v1,
               out_v0, out_v1, isem, wsem, osem, dsem):
    wid = _worker()
    idx_b = (idx_v0, idx_v1)
    w_b = (w_v0, w_v1)
    out_b = (out_v0, out_v1)

    ddenom = pltpu.async_copy(denom, denom_v, dsem)

    def start(j, b):
        base = wid * EPW + j * C2
        pltpu.async_copy(seg.at[pl.ds(base, C2)], idx_b[b], isem.at[b])
        pltpu.async_copy(w.at[pl.ds(base, C2)], w_b[b], wsem.at[b])

    start(0, 0)
    start(1, 1)
    ddenom.wait()

    def pair(p, carry):
        for b in range(2):
            j = p * 2 + b
            pltpu.make_async_copy(seg.at[pl.ds(0, C2)], idx_b[b],
                                  isem.at[b]).wait()
            pltpu.make_async_copy(w.at[pl.ds(0, C2)], w_b[b],
                                  wsem.at[b]).wait()

            @pl.when(j >= 2)
            def _():
                pltpu.make_async_copy(out_b[b], out.at[pl.ds(0, C2)],
                                      osem.at[b]).wait()

            def grp(g, cc):
                for u in range(U2):
                    o = (g * U2 + u) * 16
                    idx = idx_b[b][pl.ds(o, 16)]
                    ew = jnp.exp(w_b[b][pl.ds(o, 16)])
                    d = plsc.load_gather(denom_v, [idx])
                    out_b[b][pl.ds(o, 16)] = ew / d
                return cc
            lax.fori_loop(0, G2 // U2, grp, 0)

            base = wid * EPW + j * C2
            pltpu.async_copy(out_b[b], out.at[pl.ds(base, C2)], osem.at[b])

            @pl.when(j + 2 < NCH2)
            def _():
                start(j + 2, b)
        return carry
    lax.fori_loop(0, NCH2 // 2, pair, 0)

    for b in range(2):
        pltpu.make_async_copy(out_b[b], out.at[pl.ds(0, C2)],
                              osem.at[b]).wait()


def kernel(edge_index, edge_weight):
    seg = edge_index[1]
    partial = _segment_sum(seg, edge_weight)
    denom = jnp.sum(partial.reshape(NW, NTAB), axis=0)
    wnorm = _normalize(seg, edge_weight, denom)
    return (edge_index, wnorm)
